# Initial kernel scaffold; baseline (speedup 1.0000x reference)
import jax
import jax.numpy as jnp
from jax.experimental import pallas as pl


def kernel(inputIndex, targets, tokenEmbeddingTable):
    raise NotImplementedError("write your pallas kernel here")


# speedup vs baseline: 1.8140x; 1.8140x over previous
import functools

import jax
import jax.numpy as jnp
from jax import lax
from jax.experimental import pallas as pl
from jax.experimental.pallas import tpu as pltpu
from jax.experimental.pallas import tpu_sc as plsc

_D = 8192
_B = 8192
_NC, _NS = 2, 16
_NW = _NC * _NS
_BPW = _B // _NW
_K = 8
_NCHUNK = _BPW // _K


def _body(table_hbm, idx_hbm, out_hbm, idx_v, buf, gsem):
    wid = lax.axis_index("s") * _NC + lax.axis_index("c")
    base = wid * _BPW
    pltpu.sync_copy(idx_hbm.at[pl.ds(base, _BPW)], idx_v)

    def chunk(c, carry):
        pltpu.async_copy(
            table_hbm.at[idx_v.at[pl.ds(c * _K, _K)]], buf, gsem
        ).wait()
        pltpu.sync_copy(buf, out_hbm.at[pl.ds(base + c * _K, _K)])
        return carry

    lax.fori_loop(0, _NCHUNK, chunk, 0)


@jax.jit
def _lookup(table, idx):
    mesh = plsc.VectorSubcoreMesh(core_axis_name="c", subcore_axis_name="s")
    return pl.kernel(
        _body,
        out_type=jax.ShapeDtypeStruct((_B, _D), jnp.float32),
        mesh=mesh,
        scratch_types=[
            pltpu.VMEM((_BPW,), jnp.int32),
            pltpu.VMEM((_K, _D), jnp.float32),
            pltpu.SemaphoreType.DMA,
        ],
    )(table, idx)


def kernel(inputIndex, targets, tokenEmbeddingTable):
    idx = inputIndex.reshape(-1).astype(jnp.int32)
    out = _lookup(tokenEmbeddingTable, idx)
    return out.reshape(inputIndex.shape + (tokenEmbeddingTable.shape[1],))


# speedup vs baseline: 1.9350x; 1.0667x over previous
import jax
import jax.numpy as jnp
from jax import lax
from jax.experimental import pallas as pl
from jax.experimental.pallas import tpu as pltpu
from jax.experimental.pallas import tpu_sc as plsc

_D = 8192
_B = 8192
_NC, _NS = 2, 16
_NW = _NC * _NS
_BPW = _B // _NW
_K = 4
_NCHUNK = _BPW // _K
_NPAIR = _NCHUNK // 2


def _body(table_hbm, idx_hbm, out_hbm, idx_v, buf0, buf1, gsem, ssem0, ssem1):
    wid = lax.axis_index("s") * _NC + lax.axis_index("c")
    base = wid * _BPW
    pltpu.sync_copy(idx_hbm.at[wid], idx_v)

    def gather(c, buf):
        pltpu.async_copy(table_hbm.at[idx_v.at[c]], buf, gsem)

    def gather_wait(buf):
        pltpu.make_async_copy(table_hbm.at[pl.ds(0, _K)], buf, gsem).wait()

    def scatter(c, buf, sem):
        pltpu.async_copy(buf, out_hbm.at[pl.ds(base + c * _K, _K)], sem)

    def scatter_wait(buf, sem):
        pltpu.make_async_copy(buf, out_hbm.at[pl.ds(base, _K)], sem).wait()

    gather(0, buf0)
    gather_wait(buf0)
    scatter(0, buf0, ssem0)
    gather(1, buf1)
    gather_wait(buf1)
    scatter(1, buf1, ssem1)
    scatter_wait(buf0, ssem0)
    gather(2, buf0)

    def pair(p, carry):
        c = 2 * p
        gather_wait(buf0)
        scatter(c, buf0, ssem0)
        scatter_wait(buf1, ssem1)
        gather(c + 1, buf1)
        gather_wait(buf1)
        scatter(c + 1, buf1, ssem1)
        scatter_wait(buf0, ssem0)
        gather(c + 2, buf0)
        return carry

    lax.fori_loop(1, _NPAIR - 1, pair, 0)

    c = 2 * (_NPAIR - 1)
    gather_wait(buf0)
    scatter(c, buf0, ssem0)
    scatter_wait(buf1, ssem1)
    gather(c + 1, buf1)
    gather_wait(buf1)
    scatter(c + 1, buf1, ssem1)
    scatter_wait(buf0, ssem0)
    scatter_wait(buf1, ssem1)


@jax.jit
def _lookup(table, idx):
    mesh = plsc.VectorSubcoreMesh(core_axis_name="c", subcore_axis_name="s")
    return pl.kernel(
        _body,
        out_type=jax.ShapeDtypeStruct((_B, _D), jnp.float32),
        mesh=mesh,
        scratch_types=[
            pltpu.VMEM((_NCHUNK, _K), jnp.int32),
            pltpu.VMEM((_K, _D), jnp.float32),
            pltpu.VMEM((_K, _D), jnp.float32),
            pltpu.SemaphoreType.DMA,
            pltpu.SemaphoreType.DMA,
            pltpu.SemaphoreType.DMA,
        ],
    )(table, idx)


def kernel(inputIndex, targets, tokenEmbeddingTable):
    idx = inputIndex.reshape(_NW, _NCHUNK, _K).astype(jnp.int32)
    out = _lookup(tokenEmbeddingTable, idx)
    return out.reshape(inputIndex.shape + (tokenEmbeddingTable.shape[1],))


# speedup vs baseline: 1.9545x; 1.0101x over previous
import jax
import jax.numpy as jnp
from jax import lax
from jax.experimental import pallas as pl
from jax.experimental.pallas import tpu as pltpu
from jax.experimental.pallas import tpu_sc as plsc

_D = 8192
_B = 8192
_NC, _NS = 2, 16
_NW = _NC * _NS
_BPW = _B // _NW
_K = 2
_NCHUNK = _BPW // _K
_NBUF = 4
_NQUAD = _NCHUNK // _NBUF


def _body(table_hbm, idx_hbm, out_hbm, idx_v,
          buf0, buf1, buf2, buf3,
          g0, g1, g2, g3, s0, s1, s2, s3):
    bufs = [buf0, buf1, buf2, buf3]
    gsems = [g0, g1, g2, g3]
    ssems = [s0, s1, s2, s3]

    wid = lax.axis_index("s") * _NC + lax.axis_index("c")
    base = wid * _BPW
    pltpu.sync_copy(idx_hbm.at[wid], idx_v)

    def gather(c, slot):
        pltpu.async_copy(table_hbm.at[idx_v.at[c]], bufs[slot], gsems[slot])

    def gather_wait(slot):
        pltpu.make_async_copy(
            table_hbm.at[pl.ds(0, _K)], bufs[slot], gsems[slot]).wait()

    def scatter(c, slot):
        pltpu.async_copy(
            bufs[slot], out_hbm.at[pl.ds(base + c * _K, _K)], ssems[slot])

    def scatter_wait(slot):
        pltpu.make_async_copy(
            bufs[slot], out_hbm.at[pl.ds(base, _K)], ssems[slot]).wait()

    gather(0, 0)
    gather(1, 1)
    gather(2, 2)
    gather_wait(0)
    scatter(0, 0)
    gather(3, 3)
    gather_wait(1)
    scatter(1, 1)
    scatter_wait(0)
    gather(4, 0)
    gather_wait(2)
    scatter(2, 2)
    scatter_wait(1)
    gather(5, 1)
    gather_wait(3)
    scatter(3, 3)
    scatter_wait(2)
    gather(6, 2)

    def quad(q, carry):
        for j in range(_NBUF):
            c = _NBUF * q + j
            slot = j
            prev = (j - 1) % _NBUF
            gather_wait(slot)
            scatter(c, slot)
            scatter_wait(prev)
            gather(c + 3, prev)
        return carry

    lax.fori_loop(1, _NQUAD - 1, quad, 0)

    cb = _NCHUNK - _NBUF
    gather_wait(0)
    scatter(cb + 0, 0)
    scatter_wait(3)
    gather(cb + 3, 3)
    gather_wait(1)
    scatter(cb + 1, 1)
    scatter_wait(0)
    gather_wait(2)
    scatter(cb + 2, 2)
    scatter_wait(1)
    gather_wait(3)
    scatter(cb + 3, 3)
    scatter_wait(2)
    scatter_wait(3)


@jax.jit
def _lookup(table, idx):
    mesh = plsc.VectorSubcoreMesh(core_axis_name="c", subcore_axis_name="s")
    return pl.kernel(
        _body,
        out_type=jax.ShapeDtypeStruct((_B, _D), jnp.float32),
        mesh=mesh,
        scratch_types=(
            [pltpu.VMEM((_NCHUNK, _K), jnp.int32)]
            + [pltpu.VMEM((_K, _D), jnp.float32)] * _NBUF
            + [pltpu.SemaphoreType.DMA] * (2 * _NBUF)
        ),
    )(table, idx)


def kernel(inputIndex, targets, tokenEmbeddingTable):
    idx = inputIndex.reshape(_NW, _NCHUNK, _K).astype(jnp.int32)
    out = _lookup(tokenEmbeddingTable, idx)
    return out.reshape(inputIndex.shape + (tokenEmbeddingTable.shape[1],))


# speedup vs baseline: 1.9548x; 1.0001x over previous
import jax
import jax.numpy as jnp
from jax import lax
from jax.experimental import pallas as pl
from jax.experimental.pallas import tpu as pltpu
from jax.experimental.pallas import tpu_sc as plsc

_D = 8192
_B = 8192
_NC, _NS = 2, 16
_NW = _NC * _NS
_BPW = _B // _NW
_K = 2
_NCHUNK = _BPW // _K
_NBUF = 4
_NQUAD = _NCHUNK // _NBUF


def _body(table_hbm, idx_hbm, out_hbm, idx_v,
          buf0, buf1, buf2, buf3,
          g0, g1, g2, g3, s0, s1, s2, s3):
    bufs = [buf0, buf1, buf2, buf3]
    gsems = [g0, g1, g2, g3]
    ssems = [s0, s1, s2, s3]

    wid = lax.axis_index("s") * _NC + lax.axis_index("c")
    base = wid * _BPW
    pltpu.sync_copy(idx_hbm.at[wid], idx_v)

    def gather(c, slot):
        pltpu.async_copy(table_hbm.at[idx_v.at[c]], bufs[slot], gsems[slot])

    def gather_wait(slot):
        pltpu.make_async_copy(
            table_hbm.at[pl.ds(0, _K)], bufs[slot], gsems[slot]).wait()

    def scatter(c, slot):
        pltpu.async_copy(
            bufs[slot], out_hbm.at[pl.ds(base + c * _K, _K)], ssems[slot])

    def scatter_wait(slot):
        pltpu.make_async_copy(
            bufs[slot], out_hbm.at[pl.ds(base, _K)], ssems[slot]).wait()

    gather(0, 0)
    gather(1, 1)
    gather(2, 2)
    gather_wait(0)
    scatter(0, 0)
    gather(3, 3)
    gather_wait(1)
    scatter(1, 1)
    scatter_wait(0)
    gather(4, 0)
    gather_wait(2)
    scatter(2, 2)
    scatter_wait(1)
    gather(5, 1)
    gather_wait(3)
    scatter(3, 3)
    scatter_wait(2)
    gather(6, 2)

    def quad(q, carry):
        for j in range(_NBUF):
            c = _NBUF * q + j
            slot = j
            prev = (j - 1) % _NBUF
            gather_wait(slot)
            scatter_wait(prev)
            gather(c + 3, prev)
            scatter(c, slot)
        return carry

    lax.fori_loop(1, _NQUAD - 1, quad, 0)

    cb = _NCHUNK - _NBUF
    gather_wait(0)
    scatter(cb + 0, 0)
    scatter_wait(3)
    gather(cb + 3, 3)
    gather_wait(1)
    scatter(cb + 1, 1)
    scatter_wait(0)
    gather_wait(2)
    scatter(cb + 2, 2)
    scatter_wait(1)
    gather_wait(3)
    scatter(cb + 3, 3)
    scatter_wait(2)
    scatter_wait(3)


@jax.jit
def _lookup(table, idx):
    mesh = plsc.VectorSubcoreMesh(core_axis_name="c", subcore_axis_name="s")
    return pl.kernel(
        _body,
        out_type=jax.ShapeDtypeStruct((_B, _D), jnp.float32),
        mesh=mesh,
        scratch_types=(
            [pltpu.VMEM((_NCHUNK, _K), jnp.int32)]
            + [pltpu.VMEM((_K, _D), jnp.float32)] * _NBUF
            + [pltpu.SemaphoreType.DMA] * (2 * _NBUF)
        ),
    )(table, idx)


def kernel(inputIndex, targets, tokenEmbeddingTable):
    idx = inputIndex.reshape(_NW, _NCHUNK, _K).astype(jnp.int32)
    out = _lookup(tokenEmbeddingTable, idx)
    return out.reshape(inputIndex.shape + (tokenEmbeddingTable.shape[1],))
